# baseline (speedup 1.0000x reference)
import math

import jax
import jax.numpy as jnp
from jax.experimental import pallas as pl
from jax.experimental.pallas import tpu as pltpu

_K = 512
_L = 64
_B = 4096
_LANES = 128
_ROWS = _B * _L // _LANES
_RB = 8

_HALF_LOG_2PI = 0.5 * math.log(2.0 * math.pi)


def _mog_kernel(z_ref, mt_ref, lvt_ref, w_ref, out_ref, a_ref, b_ref, c_ref):
    i = pl.program_id(0)

    @pl.when(i == 0)
    def _():
        lv = lvt_ref[...]
        mu = mt_ref[...]
        wv = w_ref[...]
        wmax = jnp.max(wv, axis=0, keepdims=True)
        lse_w = wmax + jnp.log(jnp.sum(jnp.exp(wv - wmax), axis=0, keepdims=True))
        lw = wv - lse_w
        a2 = -0.5 * jnp.exp(-lv)
        a_ref[...] = a2
        b_ref[...] = -2.0 * a2 * mu
        c_ref[...] = (a2 * mu * mu - 0.5 * lv - _HALF_LOG_2PI) + lw

    zb = z_ref[...]
    alpha = a_ref[...][:, None, :]
    beta = b_ref[...][:, None, :]
    gamma = c_ref[...][:, None, :]
    z3 = zb[None, :, :]
    p = gamma + z3 * (beta + z3 * alpha)
    m = jnp.max(p, axis=0)
    s = jnp.sum(jnp.exp(p - m[None, :, :]), axis=0)
    out_ref[...] = m + jnp.log(s)


def kernel(z, means, logvars, w):
    z2 = z.reshape(_ROWS, _LANES)
    mt = jnp.concatenate([means, means], axis=1)
    lvt = jnp.concatenate([logvars, logvars], axis=1)
    wc = w.reshape(_K, 1)
    out2 = pl.pallas_call(
        _mog_kernel,
        grid=(_ROWS // _RB,),
        in_specs=[
            pl.BlockSpec((_RB, _LANES), lambda i: (i, 0)),
            pl.BlockSpec((_K, _LANES), lambda i: (0, 0)),
            pl.BlockSpec((_K, _LANES), lambda i: (0, 0)),
            pl.BlockSpec((_K, 1), lambda i: (0, 0)),
        ],
        out_specs=pl.BlockSpec((_RB, _LANES), lambda i: (i, 0)),
        out_shape=jax.ShapeDtypeStruct((_ROWS, _LANES), jnp.float32),
        scratch_shapes=[
            pltpu.VMEM((_K, _LANES), jnp.float32),
            pltpu.VMEM((_K, _LANES), jnp.float32),
            pltpu.VMEM((_K, _LANES), jnp.float32),
        ],
    )(z2, mt, lvt, wc)
    return out2.reshape(_B, _L)


# speedup vs baseline: 1.8036x; 1.8036x over previous
import math

import jax
import jax.numpy as jnp
from jax.experimental import pallas as pl
from jax.experimental.pallas import tpu as pltpu

_K = 512
_L = 64
_B = 4096
_LANES = 128
_ROWS = _B * _L // _LANES
_RB = 8

_HALF_LOG_2PI = 0.5 * math.log(2.0 * math.pi)
_LOG2E = math.log2(math.e)
_LN2 = math.log(2.0)


def _mog_kernel(z_ref, mt_ref, lvt_ref, w_ref, out_ref, a_ref, b_ref, c_ref):
    i = pl.program_id(0)

    @pl.when(i == 0)
    def _():
        lv = lvt_ref[...]
        mu = mt_ref[...]
        wv = w_ref[...]
        wmax = jnp.max(wv, axis=0, keepdims=True)
        lse_w = wmax + jnp.log(jnp.sum(jnp.exp(wv - wmax), axis=0, keepdims=True))
        lw = wv - lse_w
        a2 = -0.5 * jnp.exp(-lv)
        a_ref[...] = _LOG2E * a2
        b_ref[...] = _LOG2E * (-2.0 * a2 * mu)
        c_ref[...] = _LOG2E * ((a2 * mu * mu - 0.5 * lv - _HALF_LOG_2PI) + lw)

    alpha = a_ref[...]
    beta = b_ref[...]
    gamma = c_ref[...]
    for r in range(_RB):
        zrow = z_ref[r:r + 1, :]
        p = gamma + zrow * (beta + zrow * alpha)
        m = jnp.max(p, axis=0, keepdims=True)
        s = jnp.sum(jnp.exp2(p - m), axis=0, keepdims=True)
        out_ref[r:r + 1, :] = _LN2 * (m + jnp.log2(s))


def kernel(z, means, logvars, w):
    z2 = z.reshape(_ROWS, _LANES)
    mt = jnp.concatenate([means, means], axis=1)
    lvt = jnp.concatenate([logvars, logvars], axis=1)
    wc = w.reshape(_K, 1)
    out2 = pl.pallas_call(
        _mog_kernel,
        grid=(_ROWS // _RB,),
        in_specs=[
            pl.BlockSpec((_RB, _LANES), lambda i: (i, 0)),
            pl.BlockSpec((_K, _LANES), lambda i: (0, 0)),
            pl.BlockSpec((_K, _LANES), lambda i: (0, 0)),
            pl.BlockSpec((_K, 1), lambda i: (0, 0)),
        ],
        out_specs=pl.BlockSpec((_RB, _LANES), lambda i: (i, 0)),
        out_shape=jax.ShapeDtypeStruct((_ROWS, _LANES), jnp.float32),
        scratch_shapes=[
            pltpu.VMEM((_K, _LANES), jnp.float32),
            pltpu.VMEM((_K, _LANES), jnp.float32),
            pltpu.VMEM((_K, _LANES), jnp.float32),
        ],
    )(z2, mt, lvt, wc)
    return out2.reshape(_B, _L)
